# baseline (speedup 1.0000x reference)
import functools

import jax
import jax.numpy as jnp
from jax.experimental import pallas as pl
from jax.experimental.pallas import tpu as pltpu

BATCH = 1024
NDIMS = 16
BN = 2048


def _mm_block(x_ref, w_ref, o_ref):
    o_ref[...] = jax.lax.dot_general(
        x_ref[...], w_ref[...],
        dimension_numbers=(((1,), (1,)), ((), ())),
        preferred_element_type=jnp.float32,
    )


@functools.partial(jax.jit, static_argnames=())
def kernel(x, weights):
    n = weights.shape[0]
    grid = (pl.cdiv(n, BN),)
    return pl.pallas_call(
        _mm_block,
        grid=grid,
        in_specs=[
            pl.BlockSpec((BATCH, NDIMS), lambda j: (0, 0)),
            pl.BlockSpec((BN, NDIMS), lambda j: (j, 0)),
        ],
        out_specs=pl.BlockSpec((BATCH, BN), lambda j: (0, j)),
        out_shape=jax.ShapeDtypeStruct((x.shape[0], n), jnp.float32),
        compiler_params=pltpu.CompilerParams(
            dimension_semantics=("arbitrary",),
        ),
    )(x, weights)


# speedup vs baseline: 1.0983x; 1.0983x over previous
import functools

import jax
import jax.numpy as jnp
from jax.experimental import pallas as pl
from jax.experimental.pallas import tpu as pltpu

BATCH = 1024
NDIMS = 16
BN = 2048


def _mm_block(x_ref, wt_ref, o_ref):
    o_ref[...] = jax.lax.dot_general(
        x_ref[...].astype(jnp.bfloat16), wt_ref[...].astype(jnp.bfloat16),
        dimension_numbers=(((1,), (0,)), ((), ())),
        preferred_element_type=jnp.float32,
    )


@functools.partial(jax.jit, static_argnames=())
def kernel(x, weights):
    n = weights.shape[0]
    wt = weights.T
    grid = (pl.cdiv(n, BN),)
    return pl.pallas_call(
        _mm_block,
        grid=grid,
        in_specs=[
            pl.BlockSpec((BATCH, NDIMS), lambda j: (0, 0)),
            pl.BlockSpec((NDIMS, BN), lambda j: (0, j)),
        ],
        out_specs=pl.BlockSpec((BATCH, BN), lambda j: (0, j)),
        out_shape=jax.ShapeDtypeStruct((x.shape[0], n), jnp.float32),
        compiler_params=pltpu.CompilerParams(
            dimension_semantics=("arbitrary",),
        ),
    )(x, wt)
